# Initial kernel scaffold; baseline (speedup 1.0000x reference)
import jax
import jax.numpy as jnp
from jax.experimental import pallas as pl


def kernel(inputs, weight):
    raise NotImplementedError("write your pallas kernel here")


# speedup vs baseline: 2.9844x; 2.9844x over previous
import functools

import jax
import jax.numpy as jnp
from jax.experimental import pallas as pl
from jax.experimental.pallas import tpu as pltpu

NUM_EMBEDDINGS = 1024
EMBEDDING_DIM = 64
COMMITMENT_COST = 0.25
TILE = 512


def _vq_kernel(x_ref, w_ref, x2_ref, w2_ref, q_ref, enc_ref, idx_ref,
               loss_ref, ppl_ref, loss_acc, cnt_acc, *, n_tok, n_steps):
    i = pl.program_id(0)

    @pl.when(i == 0)
    def _init():
        loss_acc[0, 0] = 0.0
        cnt_acc[...] = jnp.zeros_like(cnt_acc)

    x = x_ref[...]
    w = w_ref[...]
    mm = jax.lax.dot_general(
        x, w, dimension_numbers=(((1,), (1,)), ((), ())),
        preferred_element_type=jnp.float32)
    d = (x2_ref[...] + w2_ref[...]) - 2.0 * mm
    dmin = jnp.min(d, axis=1, keepdims=True)
    iota = jax.lax.broadcasted_iota(jnp.int32, (TILE, NUM_EMBEDDINGS), 1)
    idx = jnp.min(jnp.where(d == dmin, iota, NUM_EMBEDDINGS), axis=1)
    onehot = (iota == idx[:, None]).astype(jnp.float32)
    q = jax.lax.dot_general(
        onehot, w, dimension_numbers=(((1,), (0,)), ((), ())),
        preferred_element_type=jnp.float32)

    enc_ref[...] = onehot
    q_ref[...] = q
    idx_ref[...] = idx[:, None]

    diff = q - x
    loss_acc[0, 0] += jnp.sum(diff * diff)
    cnt_acc[...] += jnp.sum(onehot, axis=0, keepdims=True)

    @pl.when(i == n_steps - 1)
    def _fin():
        n_elems = n_tok * EMBEDDING_DIM
        loss_ref[...] = jnp.full(
            (1, 1), (1.0 + COMMITMENT_COST) * loss_acc[0, 0] / n_elems)
        avg = cnt_acc[...] * (1.0 / n_tok)
        ppl_ref[...] = jnp.exp(
            -jnp.sum(avg * jnp.log(avg + 1e-10), keepdims=True))


@jax.jit
def kernel(inputs, weight):
    bs, seq_len, feat = inputs.shape
    flat = inputs.reshape(-1, EMBEDDING_DIM)
    n_tok = flat.shape[0]
    n_steps = n_tok // TILE
    x2 = jnp.sum(flat ** 2, axis=1, keepdims=True)
    w2 = jnp.sum(weight ** 2, axis=1)[None, :]

    q, enc, idx, loss, ppl = pl.pallas_call(
        functools.partial(_vq_kernel, n_tok=n_tok, n_steps=n_steps),
        grid=(n_steps,),
        in_specs=[
            pl.BlockSpec((TILE, EMBEDDING_DIM), lambda i: (i, 0)),
            pl.BlockSpec((NUM_EMBEDDINGS, EMBEDDING_DIM), lambda i: (0, 0)),
            pl.BlockSpec((TILE, 1), lambda i: (i, 0)),
            pl.BlockSpec((1, NUM_EMBEDDINGS), lambda i: (0, 0)),
        ],
        out_specs=[
            pl.BlockSpec((TILE, EMBEDDING_DIM), lambda i: (i, 0)),
            pl.BlockSpec((TILE, NUM_EMBEDDINGS), lambda i: (i, 0)),
            pl.BlockSpec((TILE, 1), lambda i: (i, 0)),
            pl.BlockSpec((1, 1), lambda i: (0, 0)),
            pl.BlockSpec((1, 1), lambda i: (0, 0)),
        ],
        out_shape=[
            jax.ShapeDtypeStruct((n_tok, EMBEDDING_DIM), jnp.float32),
            jax.ShapeDtypeStruct((n_tok, NUM_EMBEDDINGS), jnp.float32),
            jax.ShapeDtypeStruct((n_tok, 1), jnp.int32),
            jax.ShapeDtypeStruct((1, 1), jnp.float32),
            jax.ShapeDtypeStruct((1, 1), jnp.float32),
        ],
        scratch_shapes=[
            pltpu.SMEM((1, 1), jnp.float32),
            pltpu.VMEM((1, NUM_EMBEDDINGS), jnp.float32),
        ],
    )(flat, weight, x2, w2)

    return (loss[0, 0], q.reshape(bs, seq_len, feat), ppl[0, 0], enc, idx)


# speedup vs baseline: 3.2515x; 1.0895x over previous
import functools

import jax
import jax.numpy as jnp
from jax.experimental import pallas as pl
from jax.experimental.pallas import tpu as pltpu

NUM_EMBEDDINGS = 1024
EMBEDDING_DIM = 64
COMMITMENT_COST = 0.25
TILE = 1024


def _vq_kernel(x_ref, w_ref, x2_ref, w2_ref, q_ref, enc_ref, idx_ref,
               loss_ref, ppl_ref, loss_acc, cnt_acc, *, n_tok, n_steps):
    i = pl.program_id(0)

    @pl.when(i == 0)
    def _init():
        loss_acc[0, 0] = 0.0
        cnt_acc[...] = jnp.zeros_like(cnt_acc)

    x = x_ref[...]
    w = w_ref[...]
    mm = jax.lax.dot_general(
        x, w, dimension_numbers=(((1,), (1,)), ((), ())),
        preferred_element_type=jnp.float32)
    d = (x2_ref[...] + w2_ref[...]) - 2.0 * mm
    dmin = jnp.min(d, axis=1, keepdims=True)
    iota = jax.lax.broadcasted_iota(jnp.int32, (TILE, NUM_EMBEDDINGS), 1)
    idx = jnp.min(jnp.where(d == dmin, iota, NUM_EMBEDDINGS), axis=1)
    onehot = (iota == idx[:, None]).astype(jnp.float32)
    q = jax.lax.dot_general(
        onehot, w, dimension_numbers=(((1,), (0,)), ((), ())),
        preferred_element_type=jnp.float32)

    enc_ref[...] = onehot
    q_ref[...] = q
    idx_ref[...] = idx[:, None]

    diff = q - x
    loss_acc[0, 0] += jnp.sum(diff * diff)
    cnt_acc[...] += jnp.sum(onehot, axis=0, keepdims=True)

    @pl.when(i == n_steps - 1)
    def _fin():
        n_elems = n_tok * EMBEDDING_DIM
        loss_ref[...] = jnp.full(
            (1, 1), (1.0 + COMMITMENT_COST) * loss_acc[0, 0] / n_elems)
        avg = cnt_acc[...] * (1.0 / n_tok)
        ppl_ref[...] = jnp.exp(
            -jnp.sum(avg * jnp.log(avg + 1e-10), keepdims=True))


@jax.jit
def kernel(inputs, weight):
    bs, seq_len, feat = inputs.shape
    flat = inputs.reshape(-1, EMBEDDING_DIM)
    n_tok = flat.shape[0]
    n_steps = n_tok // TILE
    x2 = jnp.sum(flat ** 2, axis=1, keepdims=True)
    w2 = jnp.sum(weight ** 2, axis=1)[None, :]

    q, enc, idx, loss, ppl = pl.pallas_call(
        functools.partial(_vq_kernel, n_tok=n_tok, n_steps=n_steps),
        grid=(n_steps,),
        in_specs=[
            pl.BlockSpec((TILE, EMBEDDING_DIM), lambda i: (i, 0)),
            pl.BlockSpec((NUM_EMBEDDINGS, EMBEDDING_DIM), lambda i: (0, 0)),
            pl.BlockSpec((TILE, 1), lambda i: (i, 0)),
            pl.BlockSpec((1, NUM_EMBEDDINGS), lambda i: (0, 0)),
        ],
        out_specs=[
            pl.BlockSpec((TILE, EMBEDDING_DIM), lambda i: (i, 0)),
            pl.BlockSpec((TILE, NUM_EMBEDDINGS), lambda i: (i, 0)),
            pl.BlockSpec((TILE, 1), lambda i: (i, 0)),
            pl.BlockSpec((1, 1), lambda i: (0, 0)),
            pl.BlockSpec((1, 1), lambda i: (0, 0)),
        ],
        out_shape=[
            jax.ShapeDtypeStruct((n_tok, EMBEDDING_DIM), jnp.float32),
            jax.ShapeDtypeStruct((n_tok, NUM_EMBEDDINGS), jnp.float32),
            jax.ShapeDtypeStruct((n_tok, 1), jnp.int32),
            jax.ShapeDtypeStruct((1, 1), jnp.float32),
            jax.ShapeDtypeStruct((1, 1), jnp.float32),
        ],
        scratch_shapes=[
            pltpu.SMEM((1, 1), jnp.float32),
            pltpu.VMEM((1, NUM_EMBEDDINGS), jnp.float32),
        ],
    )(flat, weight, x2, w2)

    return (loss[0, 0], q.reshape(bs, seq_len, feat), ppl[0, 0], enc, idx)


# speedup vs baseline: 3.3840x; 1.0408x over previous
import functools

import jax
import jax.numpy as jnp
from jax.experimental import pallas as pl
from jax.experimental.pallas import tpu as pltpu

NUM_EMBEDDINGS = 1024
EMBEDDING_DIM = 64
COMMITMENT_COST = 0.25
TILE = 2048


def _vq_kernel(x_ref, w_ref, x2_ref, w2_ref, q_ref, enc_ref, idx_ref,
               loss_ref, ppl_ref, loss_acc, cnt_acc, *, n_tok, n_steps):
    i = pl.program_id(0)

    @pl.when(i == 0)
    def _init():
        loss_acc[0, 0] = 0.0
        cnt_acc[...] = jnp.zeros_like(cnt_acc)

    x = x_ref[...]
    w = w_ref[...]
    mm = jax.lax.dot_general(
        x, w, dimension_numbers=(((1,), (1,)), ((), ())),
        preferred_element_type=jnp.float32)
    d = (x2_ref[...] + w2_ref[...]) - 2.0 * mm
    dmin = jnp.min(d, axis=1, keepdims=True)
    iota = jax.lax.broadcasted_iota(jnp.int32, (TILE, NUM_EMBEDDINGS), 1)
    idx = jnp.min(jnp.where(d == dmin, iota, NUM_EMBEDDINGS), axis=1)
    onehot = (iota == idx[:, None]).astype(jnp.float32)
    q = jax.lax.dot_general(
        onehot, w, dimension_numbers=(((1,), (0,)), ((), ())),
        preferred_element_type=jnp.float32)

    enc_ref[...] = onehot
    q_ref[...] = q
    idx_ref[...] = idx[:, None]

    diff = q - x
    loss_acc[0, 0] += jnp.sum(diff * diff)
    cnt_acc[...] += jnp.sum(onehot, axis=0, keepdims=True)

    @pl.when(i == n_steps - 1)
    def _fin():
        n_elems = n_tok * EMBEDDING_DIM
        loss_ref[...] = jnp.full(
            (1, 1), (1.0 + COMMITMENT_COST) * loss_acc[0, 0] / n_elems)
        avg = cnt_acc[...] * (1.0 / n_tok)
        ppl_ref[...] = jnp.exp(
            -jnp.sum(avg * jnp.log(avg + 1e-10), keepdims=True))


@jax.jit
def kernel(inputs, weight):
    bs, seq_len, feat = inputs.shape
    flat = inputs.reshape(-1, EMBEDDING_DIM)
    n_tok = flat.shape[0]
    n_steps = n_tok // TILE
    x2 = jnp.sum(flat ** 2, axis=1, keepdims=True)
    w2 = jnp.sum(weight ** 2, axis=1)[None, :]

    q, enc, idx, loss, ppl = pl.pallas_call(
        functools.partial(_vq_kernel, n_tok=n_tok, n_steps=n_steps),
        grid=(n_steps,),
        in_specs=[
            pl.BlockSpec((TILE, EMBEDDING_DIM), lambda i: (i, 0)),
            pl.BlockSpec((NUM_EMBEDDINGS, EMBEDDING_DIM), lambda i: (0, 0)),
            pl.BlockSpec((TILE, 1), lambda i: (i, 0)),
            pl.BlockSpec((1, NUM_EMBEDDINGS), lambda i: (0, 0)),
        ],
        out_specs=[
            pl.BlockSpec((TILE, EMBEDDING_DIM), lambda i: (i, 0)),
            pl.BlockSpec((TILE, NUM_EMBEDDINGS), lambda i: (i, 0)),
            pl.BlockSpec((TILE, 1), lambda i: (i, 0)),
            pl.BlockSpec((1, 1), lambda i: (0, 0)),
            pl.BlockSpec((1, 1), lambda i: (0, 0)),
        ],
        out_shape=[
            jax.ShapeDtypeStruct((n_tok, EMBEDDING_DIM), jnp.float32),
            jax.ShapeDtypeStruct((n_tok, NUM_EMBEDDINGS), jnp.float32),
            jax.ShapeDtypeStruct((n_tok, 1), jnp.int32),
            jax.ShapeDtypeStruct((1, 1), jnp.float32),
            jax.ShapeDtypeStruct((1, 1), jnp.float32),
        ],
        scratch_shapes=[
            pltpu.SMEM((1, 1), jnp.float32),
            pltpu.VMEM((1, NUM_EMBEDDINGS), jnp.float32),
        ],
    )(flat, weight, x2, w2)

    return (loss[0, 0], q.reshape(bs, seq_len, feat), ppl[0, 0], enc, idx)
